# baseline (speedup 1.0000x reference)
import functools

import jax
import jax.numpy as jnp
from jax import lax
from jax.experimental import pallas as pl
from jax.experimental.pallas import tpu as pltpu
from jax.experimental.pallas import tpu_sc as plsc

E = 320000
N = 10000
NPAD = 10240
H = 64
NW = 32
EPW = E // NW
CH = 80
NCHUNK = EPW // CH
RPT = NPAD // 16

@functools.cache
def _mesh():
    return plsc.VectorSubcoreMesh(core_axis_name="c", subcore_axis_name="s")


def _sc_prep_body(s_hbm, r_hbm, z1_hbm, degp_hbm, dstp_hbm,
                  s_v, r_v, w_v, d_v, deg_sh):
    c = lax.axis_index("c")
    t = lax.axis_index("s")
    wid = c * 16 + t
    pltpu.sync_copy(z1_hbm.at[pl.ds(t * RPT, RPT)], deg_sh.at[pl.ds(t * RPT, RPT)])
    plsc.subcore_barrier()

    @pl.loop(0, NCHUNK)
    def _chunk(i):
        base = wid * EPW + i * CH
        pltpu.sync_copy(s_hbm.at[pl.ds(base, CH)], s_v)
        pltpu.sync_copy(r_hbm.at[pl.ds(base, CH)], r_v)

        @pl.loop(0, CH // 16)
        def _sub(j):
            sv = s_v[pl.ds(j * 16, 16)]
            rv = r_v[pl.ds(j * 16, 16)]
            neq = sv != rv
            w_v[pl.ds(j * 16, 16)] = jnp.where(neq, 1.0, 0.0).astype(jnp.float32)
            d_v[pl.ds(j * 16, 16)] = jnp.where(neq, rv, N)

        pltpu.sync_copy(d_v, dstp_hbm.at[pl.ds(base, CH)])
        pltpu.sync_copy(w_v, deg_sh.at[s_v], add=True)

    plsc.subcore_barrier()
    pltpu.sync_copy(deg_sh.at[pl.ds(t * RPT, RPT)], degp_hbm.at[c, pl.ds(t * RPT, RPT)])


def _sc_prep(s, r, z1):
    return pl.kernel(
        _sc_prep_body,
        out_type=(jax.ShapeDtypeStruct((2, NPAD), jnp.float32),
                  jax.ShapeDtypeStruct((E,), jnp.int32)),
        mesh=_mesh(),
        compiler_params=pltpu.CompilerParams(use_tc_tiling_on_sc=False),
        scratch_types=[pltpu.VMEM((CH,), jnp.int32),
                       pltpu.VMEM((CH,), jnp.int32),
                       pltpu.VMEM((CH,), jnp.float32),
                       pltpu.VMEM((CH,), jnp.int32),
                       pltpu.VMEM_SHARED((NPAD,), jnp.float32)],
    )(s, r, z1)


def _sc_scatter2_body(xe_hbm, s_hbm, r_hbm, z2_hbm, out_hbm,
                      s_v, r_v, rows_v, acc_sh):
    c = lax.axis_index("c")
    t = lax.axis_index("s")
    wid = c * 16 + t
    pltpu.sync_copy(z2_hbm.at[pl.ds(t * RPT, RPT)], acc_sh.at[pl.ds(t * RPT, RPT)])
    plsc.subcore_barrier()

    @pl.loop(0, NCHUNK)
    def _chunk(i):
        base = wid * EPW + i * CH
        pltpu.sync_copy(s_hbm.at[pl.ds(base, CH)], s_v)
        pltpu.sync_copy(r_hbm.at[pl.ds(base, CH)], r_v)
        pltpu.sync_copy(xe_hbm.at[pl.ds(base, CH)], rows_v)
        pltpu.sync_copy(rows_v, acc_sh.at[s_v], add=True)
        pltpu.sync_copy(rows_v, acc_sh.at[r_v], add=True)

    plsc.subcore_barrier()
    pltpu.sync_copy(acc_sh.at[pl.ds(t * RPT, RPT)], out_hbm.at[c, pl.ds(t * RPT, RPT)])


def _sc_scatter2(xe, s, r, z2):
    return pl.kernel(
        _sc_scatter2_body,
        out_type=jax.ShapeDtypeStruct((2, NPAD, H), jnp.float32),
        mesh=_mesh(),
        compiler_params=pltpu.CompilerParams(use_tc_tiling_on_sc=False),
        scratch_types=[pltpu.VMEM((CH,), jnp.int32),
                       pltpu.VMEM((CH,), jnp.int32),
                       pltpu.VMEM((CH, H), jnp.float32),
                       pltpu.VMEM_SHARED((NPAD, H), jnp.float32)],
    )(xe, s, r, z2)


def _sc_gather1_body(v_hbm, s_hbm, g_hbm, s_v, rows_v, sem):
    c = lax.axis_index("c")
    t = lax.axis_index("s")
    wid = c * 16 + t

    @pl.loop(0, NCHUNK)
    def _chunk(i):
        base = wid * EPW + i * CH
        pltpu.sync_copy(s_hbm.at[pl.ds(base, CH)], s_v)
        pltpu.async_copy(v_hbm.at[s_v], rows_v, sem).wait()
        pltpu.sync_copy(rows_v, g_hbm.at[pl.ds(base, CH)])


def _sc_gather1(v, s):
    return pl.kernel(
        _sc_gather1_body,
        out_type=jax.ShapeDtypeStruct((E, H), jnp.float32),
        mesh=_mesh(),
        compiler_params=pltpu.CompilerParams(use_tc_tiling_on_sc=False),
        scratch_types=[pltpu.VMEM((CH,), jnp.int32),
                       pltpu.VMEM((CH, H), jnp.float32),
                       pltpu.SemaphoreType.DMA],
    )(v, s)


def _sc_scatter1_body(val_hbm, d_hbm, z2_hbm, out_hbm, d_v, rows_v, acc_sh):
    c = lax.axis_index("c")
    t = lax.axis_index("s")
    wid = c * 16 + t
    pltpu.sync_copy(z2_hbm.at[pl.ds(t * RPT, RPT)], acc_sh.at[pl.ds(t * RPT, RPT)])
    plsc.subcore_barrier()

    @pl.loop(0, NCHUNK)
    def _chunk(i):
        base = wid * EPW + i * CH
        pltpu.sync_copy(d_hbm.at[pl.ds(base, CH)], d_v)
        pltpu.sync_copy(val_hbm.at[pl.ds(base, CH)], rows_v)
        pltpu.sync_copy(rows_v, acc_sh.at[d_v], add=True)

    plsc.subcore_barrier()
    pltpu.sync_copy(acc_sh.at[pl.ds(t * RPT, RPT)], out_hbm.at[c, pl.ds(t * RPT, RPT)])


def _sc_scatter1(val, dstp, z2):
    return pl.kernel(
        _sc_scatter1_body,
        out_type=jax.ShapeDtypeStruct((2, NPAD, H), jnp.float32),
        mesh=_mesh(),
        compiler_params=pltpu.CompilerParams(use_tc_tiling_on_sc=False),
        scratch_types=[pltpu.VMEM((CH,), jnp.int32),
                       pltpu.VMEM((CH, H), jnp.float32),
                       pltpu.VMEM_SHARED((NPAD, H), jnp.float32)],
    )(val, dstp, z2)


def _sc_gather2_body(xn_hbm, s_hbm, r_hbm, gs_hbm, gr_hbm,
                     s_v, r_v, rows_v, sem):
    c = lax.axis_index("c")
    t = lax.axis_index("s")
    wid = c * 16 + t

    @pl.loop(0, NCHUNK)
    def _chunk(i):
        base = wid * EPW + i * CH
        pltpu.sync_copy(s_hbm.at[pl.ds(base, CH)], s_v)
        pltpu.sync_copy(r_hbm.at[pl.ds(base, CH)], r_v)
        pltpu.async_copy(xn_hbm.at[s_v], rows_v, sem).wait()
        pltpu.sync_copy(rows_v, gs_hbm.at[pl.ds(base, CH)])
        pltpu.async_copy(xn_hbm.at[r_v], rows_v, sem).wait()
        pltpu.sync_copy(rows_v, gr_hbm.at[pl.ds(base, CH)])


def _sc_gather2(xn, s, r):
    return pl.kernel(
        _sc_gather2_body,
        out_type=(jax.ShapeDtypeStruct((E, H), jnp.float32),
                  jax.ShapeDtypeStruct((E, H), jnp.float32)),
        mesh=_mesh(),
        compiler_params=pltpu.CompilerParams(use_tc_tiling_on_sc=False),
        scratch_types=[pltpu.VMEM((CH,), jnp.int32),
                       pltpu.VMEM((CH,), jnp.int32),
                       pltpu.VMEM((CH, H), jnp.float32),
                       pltpu.SemaphoreType.DMA],
    )(xn, s, r)


_F32 = functools.partial(jnp.dot, preferred_element_type=jnp.float32)
EB = 2000
NB = 2048


def _xsum64(h):
    p = h[:, 0:8] + h[:, 8:16]
    for k in range(2, 8):
        p = p + h[:, 8 * k:8 * k + 8]
    q = p[:, 0:4] + p[:, 4:8]
    r = q[:, 0:2] + q[:, 2:4]
    return r[:, 0:1] + r[:, 1:2]


def _layer_norm(h, g, b, width=64.0):
    if width == 3.0:
        mu = ((h[:, 0:1] + h[:, 1:2]) + h[:, 2:3]) / 3.0
        d = h - mu
        var = ((d[:, 0:1] * d[:, 0:1] + d[:, 1:2] * d[:, 1:2])
               + d[:, 2:3] * d[:, 2:3]) / 3.0
    else:
        mu = _xsum64(h) / 64.0
        d = h - mu
        var = _xsum64(d * d) / 64.0
    return d * lax.rsqrt(var + 1e-5) * g + b


def _mlp_tail(h0, w1, b1, w2, b2, w3, b3, g, b, width=float(H)):
    h = jnp.maximum(h0, 0.0)
    h = jnp.maximum(_F32(h, w1[...]) + b1[...], 0.0)
    h = jnp.maximum(_F32(h, w2[...]) + b2[...], 0.0)
    h = _F32(h, w3[...]) + b3[...]
    return _layer_norm(h, g[...], b[...], width)


def _tc_mlp_body(x, w0, b0, w1, b1, w2, b2, w3, b3, g, b, o):
    h0 = _F32(x[...], w0[...]) + b0[...]
    o[...] = _mlp_tail(h0, w1, b1, w2, b2, w3, b3, g, b)


def _tc_mlp(p, x, win):
    nblk = x.shape[0] // EB
    args = (x, p["l0"]["w"], p["l0"]["b"].reshape(1, H),
            p["l1"]["w"], p["l1"]["b"].reshape(1, H),
            p["l2"]["w"], p["l2"]["b"].reshape(1, H),
            p["l3"]["w"], p["l3"]["b"].reshape(1, H),
            p["ln_g"].reshape(1, H), p["ln_b"].reshape(1, H))
    wspec = [pl.BlockSpec(a.shape, lambda i: (0, 0)) for a in args[1:]]
    return pl.pallas_call(
        _tc_mlp_body,
        grid=(nblk,),
        in_specs=[pl.BlockSpec((EB, win), lambda i: (i, 0))] + wspec,
        out_specs=pl.BlockSpec((EB, H), lambda i: (i, 0)),
        out_shape=jax.ShapeDtypeStruct((x.shape[0], H), jnp.float32),
    )(*args)


def _tc_dec_body(gs, gr, xe, w0, b0, w1, b1, w2, b2, w3, b3, g, b, o):
    cat = jnp.concatenate([gs[...], gr[...], xe[...]], axis=1)
    h0 = _F32(cat, w0[...]) + b0[...]
    o[...] = _mlp_tail(h0, w1, b1, w2, b2, w3, b3, g, b)


def _tc_dec_final_body(gs, gr, xe, w0, b0, w1, b1, w2, b2, w3, b3, g, b, o):
    cat = jnp.concatenate([gs[...], gr[...], xe[...]], axis=1)
    h0 = _F32(cat, w0[...]) + b0[...]
    o[...] = _mlp_tail(h0, w1, b1, w2, b2, w3, b3, g, b, width=3.0)


def _pad8(a):
    out = jnp.zeros(a.shape[:-1] + (8,), a.dtype)
    return out.at[..., :a.shape[-1]].set(a)


def _tc_dec(p, gs, gr, xe, final=False):
    w0 = p["l0"]["w"]
    if final:
        rest = (_pad8(w0), _pad8(p["l0"]["b"]).reshape(1, 8),
                _pad8(_pad8(p["l1"]["w"]).T).T, _pad8(p["l1"]["b"]).reshape(1, 8),
                _pad8(_pad8(p["l2"]["w"]).T).T, _pad8(p["l2"]["b"]).reshape(1, 8),
                _pad8(_pad8(p["l3"]["w"]).T).T, _pad8(p["l3"]["b"]).reshape(1, 8),
                _pad8(p["ln_g"]).reshape(1, 8), _pad8(p["ln_b"]).reshape(1, 8))
        width, body = 8, _tc_dec_final_body
    else:
        rest = (w0, p["l0"]["b"].reshape(1, H),
                p["l1"]["w"], p["l1"]["b"].reshape(1, H),
                p["l2"]["w"], p["l2"]["b"].reshape(1, H),
                p["l3"]["w"], p["l3"]["b"].reshape(1, H),
                p["ln_g"].reshape(1, H), p["ln_b"].reshape(1, H))
        width, body = H, _tc_dec_body
    args = (gs, gr, xe) + rest
    wspec = [pl.BlockSpec(a.shape, lambda i: (0, 0)) for a in args[3:]]
    return pl.pallas_call(
        body,
        grid=(E // EB,),
        in_specs=[pl.BlockSpec((EB, H), lambda i: (i, 0))] * 3 + wspec,
        out_specs=pl.BlockSpec((EB, width), lambda i: (i, 0)),
        out_shape=jax.ShapeDtypeStruct((E, width), jnp.float32),
    )(*args)


def _tc_dis_body(d0, d1, o):
    deg = d0[...] + d1[...]
    o[...] = jnp.where(deg > 0, 1.0 / jnp.sqrt(jnp.maximum(deg, 1e-12)), 0.0)


def _tc_dis(degp):
    flat = degp.reshape(2, 80, 128)
    out = pl.pallas_call(
        _tc_dis_body,
        in_specs=[pl.BlockSpec((80, 128), lambda: (0, 0))] * 2,
        out_specs=pl.BlockSpec((80, 128), lambda: (0, 0)),
        out_shape=jax.ShapeDtypeStruct((80, 128), jnp.float32),
    )(flat[0], flat[1])
    return out.reshape(NPAD, 1)


def _tc_comb0_body(p0, p1, shift, dis, xn_o, u_o):
    xn = p0[...] + p1[...] + shift[...]
    xn_o[...] = xn
    u_o[...] = xn * dis[...]


def _tc_comb0(p0, p1, shift, dis):
    return pl.pallas_call(
        _tc_comb0_body,
        grid=(NPAD // NB,),
        in_specs=[pl.BlockSpec((NB, H), lambda i: (i, 0)),
                  pl.BlockSpec((NB, H), lambda i: (i, 0)),
                  pl.BlockSpec((1, 1), lambda i: (0, 0)),
                  pl.BlockSpec((NB, 1), lambda i: (i, 0))],
        out_specs=[pl.BlockSpec((NB, H), lambda i: (i, 0))] * 2,
        out_shape=(jax.ShapeDtypeStruct((NPAD, H), jnp.float32),
                   jax.ShapeDtypeStruct((NPAD, H), jnp.float32)),
    )(p0, p1, shift, dis)


def _tc_norm_body(ds, dr, s, r, o):
    w = jnp.where(s[...] != r[...], 1.0, 0.0).astype(jnp.float32)
    o[...] = -(ds[...] * w * dr[...])


def _tc_norm(ds, dr, s, r):
    cspec = pl.BlockSpec((EB, 1), lambda i: (i, 0))
    return pl.pallas_call(
        _tc_norm_body,
        grid=(E // EB,),
        in_specs=[cspec] * 4,
        out_specs=cspec,
        out_shape=jax.ShapeDtypeStruct((E, 1), jnp.float32),
    )(ds, dr, s, r)


def _tc_scale_body(g, nrm, o):
    o[...] = nrm[...] * g[...]


def _tc_scale(g, nrm):
    return pl.pallas_call(
        _tc_scale_body,
        grid=(E // EB,),
        in_specs=[pl.BlockSpec((EB, H), lambda i: (i, 0)),
                  pl.BlockSpec((EB, 1), lambda i: (i, 0))],
        out_specs=pl.BlockSpec((EB, H), lambda i: (i, 0)),
        out_shape=jax.ShapeDtypeStruct((E, H), jnp.float32),
    )(g, nrm)


def _tc_lin1_body(p0, p1, tx_o):
    tx_o[...] = p0[...] + p1[...]


def _tc_link_body(p0, p1, txpp, tx_o):
    tx_o[...] = 2.0 * (p0[...] + p1[...]) - txpp[...]


def _tc_lin(p0, p1, txpp=None):
    body = _tc_lin1_body if txpp is None else _tc_link_body
    extra = [] if txpp is None else [txpp]
    nspec = pl.BlockSpec((NB, H), lambda i: (i, 0))
    return pl.pallas_call(
        body,
        grid=(NPAD // NB,),
        in_specs=[nspec] * (2 + len(extra)),
        out_specs=nspec,
        out_shape=jax.ShapeDtypeStruct((NPAD, H), jnp.float32),
    )(p0, p1, *extra)


def _tc_cheb_mlp_body(t0, t1, t2, t3, t4, c0, c1, c2, c3, c4, cb,
                      w0, b0, w1, b1, w2, b2, w3, b3, g, b, dis, xn_o, u_o):
    acc = (_F32(t0[...], c0[...]) + _F32(t1[...], c1[...])
           + _F32(t2[...], c2[...]) + _F32(t3[...], c3[...])
           + _F32(t4[...], c4[...]) + cb[...])
    h0 = _F32(acc, w0[...]) + b0[...]
    xn = _mlp_tail(h0, w1, b1, w2, b2, w3, b3, g, b)
    xn_o[...] = xn
    u_o[...] = xn * dis[...]


def _tc_cheb_mlp(gn, txs, dis):
    cw, m = gn["cheb"], gn["mlp"]
    args = tuple(txs) + tuple(cw["w"][k] for k in range(5)) + (
        cw["b"].reshape(1, H),
        m["l0"]["w"], m["l0"]["b"].reshape(1, H),
        m["l1"]["w"], m["l1"]["b"].reshape(1, H),
        m["l2"]["w"], m["l2"]["b"].reshape(1, H),
        m["l3"]["w"], m["l3"]["b"].reshape(1, H),
        m["ln_g"].reshape(1, H), m["ln_b"].reshape(1, H), dis)
    nspec = pl.BlockSpec((NB, H), lambda i: (i, 0))
    wspec = [pl.BlockSpec(a.shape, lambda i: (0, 0)) for a in args[5:-1]]
    return pl.pallas_call(
        _tc_cheb_mlp_body,
        grid=(NPAD // NB,),
        in_specs=[nspec] * 5 + wspec + [pl.BlockSpec((NB, 1), lambda i: (i, 0))],
        out_specs=[nspec] * 2,
        out_shape=(jax.ShapeDtypeStruct((NPAD, H), jnp.float32),
                   jax.ShapeDtypeStruct((NPAD, H), jnp.float32)),
    )(*args)


def kernel(x_edge, edge_index, num_nodes, params):
    s = edge_index[:, 0]
    r = edge_index[:, 1]
    z1 = jnp.zeros((NPAD,), jnp.float32)
    z2 = jnp.zeros((NPAD, H), jnp.float32)

    degp, dstp = _sc_prep(s, r, z1)
    dis = _tc_dis(degp)

    dis64 = jnp.broadcast_to(dis, (NPAD, H))
    gsd, grd = _sc_gather2(dis64, s, r)
    norm = _tc_norm(gsd[:, :1], grd[:, :1], s.reshape(E, 1), r.reshape(E, 1))

    def lap(v):
        g = _sc_gather1(v, s)
        val = _tc_scale(g, norm)
        return _sc_scatter1(val, dstp, z2)

    xe = _tc_mlp(params["enc"], x_edge, 16)
    p2 = _sc_scatter2(xe, s, r, z2)
    shift = (jnp.asarray(num_nodes, jnp.float32) - float(N)).reshape(1, 1)
    xn, u = _tc_comb0(p2[0], p2[1], shift, dis)

    for i in range(3):
        tx0 = xn
        p = lap(tx0)
        tx1 = _tc_lin(p[0], p[1])
        p = lap(tx1)
        tx2 = _tc_lin(p[0], p[1], tx0)
        p = lap(tx2)
        tx3 = _tc_lin(p[0], p[1], tx1)
        p = lap(tx3)
        tx4 = _tc_lin(p[0], p[1], tx2)
        xn, u = _tc_cheb_mlp(params["gn"][i], (tx0, tx1, tx2, tx3, tx4), dis)
        gs, gr = _sc_gather2(xn, s, r)
        if i < 2:
            xe = _tc_dec(params["ed"][i], gs, gr, xe)
        else:
            out = _tc_dec(params["ed"][2], gs, gr, xe, final=True)
    return out[:, :3]


# speedup vs baseline: 1.8677x; 1.8677x over previous
import functools

import jax
import jax.numpy as jnp
from jax import lax
from jax.experimental import pallas as pl
from jax.experimental.pallas import tpu as pltpu
from jax.experimental.pallas import tpu_sc as plsc

E = 320000
N = 10000
NPAD = 10240
H = 64
NW = 32
EPW = E // NW
CH = 80
NCHUNK = EPW // CH
RPT = NPAD // 16

@functools.cache
def _mesh():
    return plsc.VectorSubcoreMesh(core_axis_name="c", subcore_axis_name="s")


def _sc_prep_body(s_hbm, r_hbm, z1_hbm, degp_hbm, dstp_hbm,
                  s_v, r_v, w_v, d_v, deg_sh):
    c = lax.axis_index("c")
    t = lax.axis_index("s")
    wid = c * 16 + t
    pltpu.sync_copy(z1_hbm.at[pl.ds(t * RPT, RPT)], deg_sh.at[pl.ds(t * RPT, RPT)])
    plsc.subcore_barrier()

    @pl.loop(0, NCHUNK)
    def _chunk(i):
        base = wid * EPW + i * CH
        pltpu.sync_copy(s_hbm.at[pl.ds(base, CH)], s_v)
        pltpu.sync_copy(r_hbm.at[pl.ds(base, CH)], r_v)

        @pl.loop(0, CH // 16)
        def _sub(j):
            sv = s_v[pl.ds(j * 16, 16)]
            rv = r_v[pl.ds(j * 16, 16)]
            neq = sv != rv
            w_v[pl.ds(j * 16, 16)] = jnp.where(neq, 1.0, 0.0).astype(jnp.float32)
            d_v[pl.ds(j * 16, 16)] = jnp.where(neq, rv, N)

        pltpu.sync_copy(d_v, dstp_hbm.at[pl.ds(base, CH)])
        pltpu.sync_copy(w_v, deg_sh.at[s_v], add=True)

    plsc.subcore_barrier()
    pltpu.sync_copy(deg_sh.at[pl.ds(t * RPT, RPT)], degp_hbm.at[c, pl.ds(t * RPT, RPT)])


def _sc_prep(s, r, z1):
    return pl.kernel(
        _sc_prep_body,
        out_type=(jax.ShapeDtypeStruct((2, NPAD), jnp.float32),
                  jax.ShapeDtypeStruct((E,), jnp.int32)),
        mesh=_mesh(),
        compiler_params=pltpu.CompilerParams(use_tc_tiling_on_sc=False),
        scratch_types=[pltpu.VMEM((CH,), jnp.int32),
                       pltpu.VMEM((CH,), jnp.int32),
                       pltpu.VMEM((CH,), jnp.float32),
                       pltpu.VMEM((CH,), jnp.int32),
                       pltpu.VMEM_SHARED((NPAD,), jnp.float32)],
    )(s, r, z1)


def _sc_scatter2_body(xe_hbm, s_hbm, r_hbm, z2_hbm, out_hbm,
                      s_v, r_v, rows_v, acc_sh):
    c = lax.axis_index("c")
    t = lax.axis_index("s")
    wid = c * 16 + t
    pltpu.sync_copy(z2_hbm.at[pl.ds(t * RPT, RPT)], acc_sh.at[pl.ds(t * RPT, RPT)])
    plsc.subcore_barrier()

    @pl.loop(0, NCHUNK)
    def _chunk(i):
        base = wid * EPW + i * CH
        pltpu.sync_copy(s_hbm.at[pl.ds(base, CH)], s_v)
        pltpu.sync_copy(r_hbm.at[pl.ds(base, CH)], r_v)
        pltpu.sync_copy(xe_hbm.at[pl.ds(base, CH)], rows_v)
        pltpu.sync_copy(rows_v, acc_sh.at[s_v], add=True)
        pltpu.sync_copy(rows_v, acc_sh.at[r_v], add=True)

    plsc.subcore_barrier()
    pltpu.sync_copy(acc_sh.at[pl.ds(t * RPT, RPT)], out_hbm.at[c, pl.ds(t * RPT, RPT)])


def _sc_scatter2(xe, s, r, z2):
    return pl.kernel(
        _sc_scatter2_body,
        out_type=jax.ShapeDtypeStruct((2, NPAD, H), jnp.float32),
        mesh=_mesh(),
        compiler_params=pltpu.CompilerParams(use_tc_tiling_on_sc=False),
        scratch_types=[pltpu.VMEM((CH,), jnp.int32),
                       pltpu.VMEM((CH,), jnp.int32),
                       pltpu.VMEM((CH, H), jnp.float32),
                       pltpu.VMEM_SHARED((NPAD, H), jnp.float32)],
    )(xe, s, r, z2)


def _sc_lap_body(u_hbm, s_hbm, dstp_hbm, z2_hbm, out_hbm,
                 s_v, d_v, rows_v, acc_sh, sem):
    c = lax.axis_index("c")
    t = lax.axis_index("s")
    wid = c * 16 + t
    pltpu.sync_copy(z2_hbm.at[pl.ds(t * RPT, RPT)], acc_sh.at[pl.ds(t * RPT, RPT)])
    plsc.subcore_barrier()

    @pl.loop(0, NCHUNK)
    def _chunk(i):
        base = wid * EPW + i * CH
        pltpu.sync_copy(s_hbm.at[pl.ds(base, CH)], s_v)
        pltpu.sync_copy(dstp_hbm.at[pl.ds(base, CH)], d_v)
        pltpu.async_copy(u_hbm.at[s_v], rows_v, sem).wait()
        pltpu.sync_copy(rows_v, acc_sh.at[d_v], add=True)

    plsc.subcore_barrier()
    pltpu.sync_copy(acc_sh.at[pl.ds(t * RPT, RPT)], out_hbm.at[c, pl.ds(t * RPT, RPT)])


def _sc_lap(u, s, dstp, z2):
    return pl.kernel(
        _sc_lap_body,
        out_type=jax.ShapeDtypeStruct((2, NPAD, H), jnp.float32),
        mesh=_mesh(),
        compiler_params=pltpu.CompilerParams(use_tc_tiling_on_sc=False),
        scratch_types=[pltpu.VMEM((CH,), jnp.int32),
                       pltpu.VMEM((CH,), jnp.int32),
                       pltpu.VMEM((CH, H), jnp.float32),
                       pltpu.VMEM_SHARED((NPAD, H), jnp.float32),
                       pltpu.SemaphoreType.DMA],
    )(u, s, dstp, z2)


def _sc_gather2_body(xn_hbm, s_hbm, r_hbm, gs_hbm, gr_hbm,
                     s_v, r_v, rows_v, sem):
    c = lax.axis_index("c")
    t = lax.axis_index("s")
    wid = c * 16 + t

    @pl.loop(0, NCHUNK)
    def _chunk(i):
        base = wid * EPW + i * CH
        pltpu.sync_copy(s_hbm.at[pl.ds(base, CH)], s_v)
        pltpu.sync_copy(r_hbm.at[pl.ds(base, CH)], r_v)
        pltpu.async_copy(xn_hbm.at[s_v], rows_v, sem).wait()
        pltpu.sync_copy(rows_v, gs_hbm.at[pl.ds(base, CH)])
        pltpu.async_copy(xn_hbm.at[r_v], rows_v, sem).wait()
        pltpu.sync_copy(rows_v, gr_hbm.at[pl.ds(base, CH)])


def _sc_gather2(xn, s, r):
    return pl.kernel(
        _sc_gather2_body,
        out_type=(jax.ShapeDtypeStruct((E, H), jnp.float32),
                  jax.ShapeDtypeStruct((E, H), jnp.float32)),
        mesh=_mesh(),
        compiler_params=pltpu.CompilerParams(use_tc_tiling_on_sc=False),
        scratch_types=[pltpu.VMEM((CH,), jnp.int32),
                       pltpu.VMEM((CH,), jnp.int32),
                       pltpu.VMEM((CH, H), jnp.float32),
                       pltpu.SemaphoreType.DMA],
    )(xn, s, r)


_F32 = functools.partial(jnp.dot, preferred_element_type=jnp.float32)
EB = 2000
NB = 2048


def _xsum64(h):
    p = h[:, 0:8] + h[:, 8:16]
    for k in range(2, 8):
        p = p + h[:, 8 * k:8 * k + 8]
    q = p[:, 0:4] + p[:, 4:8]
    r = q[:, 0:2] + q[:, 2:4]
    return r[:, 0:1] + r[:, 1:2]


def _layer_norm(h, g, b, width=64.0):
    if width == 3.0:
        mu = ((h[:, 0:1] + h[:, 1:2]) + h[:, 2:3]) / 3.0
        d = h - mu
        var = ((d[:, 0:1] * d[:, 0:1] + d[:, 1:2] * d[:, 1:2])
               + d[:, 2:3] * d[:, 2:3]) / 3.0
    else:
        mu = _xsum64(h) / 64.0
        d = h - mu
        var = _xsum64(d * d) / 64.0
    return d * lax.rsqrt(var + 1e-5) * g + b


def _mlp_tail(h0, w1, b1, w2, b2, w3, b3, g, b, width=float(H)):
    h = jnp.maximum(h0, 0.0)
    h = jnp.maximum(_F32(h, w1[...]) + b1[...], 0.0)
    h = jnp.maximum(_F32(h, w2[...]) + b2[...], 0.0)
    h = _F32(h, w3[...]) + b3[...]
    return _layer_norm(h, g[...], b[...], width)


def _tc_mlp_body(x, w0, b0, w1, b1, w2, b2, w3, b3, g, b, o):
    h0 = _F32(x[...], w0[...]) + b0[...]
    o[...] = _mlp_tail(h0, w1, b1, w2, b2, w3, b3, g, b)


def _tc_mlp(p, x, win):
    nblk = x.shape[0] // EB
    args = (x, p["l0"]["w"], p["l0"]["b"].reshape(1, H),
            p["l1"]["w"], p["l1"]["b"].reshape(1, H),
            p["l2"]["w"], p["l2"]["b"].reshape(1, H),
            p["l3"]["w"], p["l3"]["b"].reshape(1, H),
            p["ln_g"].reshape(1, H), p["ln_b"].reshape(1, H))
    wspec = [pl.BlockSpec(a.shape, lambda i: (0, 0)) for a in args[1:]]
    return pl.pallas_call(
        _tc_mlp_body,
        grid=(nblk,),
        in_specs=[pl.BlockSpec((EB, win), lambda i: (i, 0))] + wspec,
        out_specs=pl.BlockSpec((EB, H), lambda i: (i, 0)),
        out_shape=jax.ShapeDtypeStruct((x.shape[0], H), jnp.float32),
    )(*args)


def _tc_dec_body(gs, gr, xe, w0, b0, w1, b1, w2, b2, w3, b3, g, b, o):
    cat = jnp.concatenate([gs[...], gr[...], xe[...]], axis=1)
    h0 = _F32(cat, w0[...]) + b0[...]
    o[...] = _mlp_tail(h0, w1, b1, w2, b2, w3, b3, g, b)


def _tc_dec_final_body(gs, gr, xe, w0, b0, w1, b1, w2, b2, w3, b3, g, b, o):
    cat = jnp.concatenate([gs[...], gr[...], xe[...]], axis=1)
    h0 = _F32(cat, w0[...]) + b0[...]
    o[...] = _mlp_tail(h0, w1, b1, w2, b2, w3, b3, g, b, width=3.0)


def _pad8(a):
    out = jnp.zeros(a.shape[:-1] + (8,), a.dtype)
    return out.at[..., :a.shape[-1]].set(a)


def _tc_dec(p, gs, gr, xe, final=False):
    w0 = p["l0"]["w"]
    if final:
        rest = (_pad8(w0), _pad8(p["l0"]["b"]).reshape(1, 8),
                _pad8(_pad8(p["l1"]["w"]).T).T, _pad8(p["l1"]["b"]).reshape(1, 8),
                _pad8(_pad8(p["l2"]["w"]).T).T, _pad8(p["l2"]["b"]).reshape(1, 8),
                _pad8(_pad8(p["l3"]["w"]).T).T, _pad8(p["l3"]["b"]).reshape(1, 8),
                _pad8(p["ln_g"]).reshape(1, 8), _pad8(p["ln_b"]).reshape(1, 8))
        width, body = 8, _tc_dec_final_body
    else:
        rest = (w0, p["l0"]["b"].reshape(1, H),
                p["l1"]["w"], p["l1"]["b"].reshape(1, H),
                p["l2"]["w"], p["l2"]["b"].reshape(1, H),
                p["l3"]["w"], p["l3"]["b"].reshape(1, H),
                p["ln_g"].reshape(1, H), p["ln_b"].reshape(1, H))
        width, body = H, _tc_dec_body
    args = (gs, gr, xe) + rest
    wspec = [pl.BlockSpec(a.shape, lambda i: (0, 0)) for a in args[3:]]
    return pl.pallas_call(
        body,
        grid=(E // EB,),
        in_specs=[pl.BlockSpec((EB, H), lambda i: (i, 0))] * 3 + wspec,
        out_specs=pl.BlockSpec((EB, width), lambda i: (i, 0)),
        out_shape=jax.ShapeDtypeStruct((E, width), jnp.float32),
    )(*args)


def _tc_dis_body(d0, d1, o):
    deg = d0[...] + d1[...]
    o[...] = jnp.where(deg > 0, 1.0 / jnp.sqrt(jnp.maximum(deg, 1e-12)), 0.0)


def _tc_dis(degp):
    flat = degp.reshape(2, 80, 128)
    out = pl.pallas_call(
        _tc_dis_body,
        in_specs=[pl.BlockSpec((80, 128), lambda: (0, 0))] * 2,
        out_specs=pl.BlockSpec((80, 128), lambda: (0, 0)),
        out_shape=jax.ShapeDtypeStruct((80, 128), jnp.float32),
    )(flat[0], flat[1])
    return out.reshape(NPAD, 1)


def _tc_comb0_body(p0, p1, shift, dis, xn_o, u_o):
    xn = p0[...] + p1[...] + shift[...]
    xn_o[...] = xn
    u_o[...] = xn * dis[...]


def _tc_comb0(p0, p1, shift, dis):
    return pl.pallas_call(
        _tc_comb0_body,
        grid=(NPAD // NB,),
        in_specs=[pl.BlockSpec((NB, H), lambda i: (i, 0)),
                  pl.BlockSpec((NB, H), lambda i: (i, 0)),
                  pl.BlockSpec((1, 1), lambda i: (0, 0)),
                  pl.BlockSpec((NB, 1), lambda i: (i, 0))],
        out_specs=[pl.BlockSpec((NB, H), lambda i: (i, 0))] * 2,
        out_shape=(jax.ShapeDtypeStruct((NPAD, H), jnp.float32),
                   jax.ShapeDtypeStruct((NPAD, H), jnp.float32)),
    )(p0, p1, shift, dis)


def _tc_hop1_body(p0, p1, dis, tx_o, u_o):
    t = -(p0[...] + p1[...]) * dis[...]
    tx_o[...] = t
    u_o[...] = t * dis[...]


def _tc_hopk_body(p0, p1, txpp, dis, tx_o, u_o):
    t = -2.0 * (p0[...] + p1[...]) * dis[...] - txpp[...]
    tx_o[...] = t
    u_o[...] = t * dis[...]


def _tc_hop(p0, p1, dis, txpp=None):
    body = _tc_hop1_body if txpp is None else _tc_hopk_body
    extra = [] if txpp is None else [txpp]
    nspec = pl.BlockSpec((NB, H), lambda i: (i, 0))
    return pl.pallas_call(
        body,
        grid=(NPAD // NB,),
        in_specs=[nspec, nspec] + [nspec] * len(extra)
        + [pl.BlockSpec((NB, 1), lambda i: (i, 0))],
        out_specs=[nspec] * 2,
        out_shape=(jax.ShapeDtypeStruct((NPAD, H), jnp.float32),
                   jax.ShapeDtypeStruct((NPAD, H), jnp.float32)),
    )(p0, p1, *extra, dis)


def _tc_cheb_mlp_body(t0, t1, t2, t3, t4, c0, c1, c2, c3, c4, cb,
                      w0, b0, w1, b1, w2, b2, w3, b3, g, b, dis, xn_o, u_o):
    acc = (_F32(t0[...], c0[...]) + _F32(t1[...], c1[...])
           + _F32(t2[...], c2[...]) + _F32(t3[...], c3[...])
           + _F32(t4[...], c4[...]) + cb[...])
    h0 = _F32(acc, w0[...]) + b0[...]
    xn = _mlp_tail(h0, w1, b1, w2, b2, w3, b3, g, b)
    xn_o[...] = xn
    u_o[...] = xn * dis[...]


def _tc_cheb_mlp(gn, txs, dis):
    cw, m = gn["cheb"], gn["mlp"]
    args = tuple(txs) + tuple(cw["w"][k] for k in range(5)) + (
        cw["b"].reshape(1, H),
        m["l0"]["w"], m["l0"]["b"].reshape(1, H),
        m["l1"]["w"], m["l1"]["b"].reshape(1, H),
        m["l2"]["w"], m["l2"]["b"].reshape(1, H),
        m["l3"]["w"], m["l3"]["b"].reshape(1, H),
        m["ln_g"].reshape(1, H), m["ln_b"].reshape(1, H), dis)
    nspec = pl.BlockSpec((NB, H), lambda i: (i, 0))
    wspec = [pl.BlockSpec(a.shape, lambda i: (0, 0)) for a in args[5:-1]]
    return pl.pallas_call(
        _tc_cheb_mlp_body,
        grid=(NPAD // NB,),
        in_specs=[nspec] * 5 + wspec + [pl.BlockSpec((NB, 1), lambda i: (i, 0))],
        out_specs=[nspec] * 2,
        out_shape=(jax.ShapeDtypeStruct((NPAD, H), jnp.float32),
                   jax.ShapeDtypeStruct((NPAD, H), jnp.float32)),
    )(*args)


def kernel(x_edge, edge_index, num_nodes, params):
    s = edge_index[:, 0]
    r = edge_index[:, 1]
    z1 = jnp.zeros((NPAD,), jnp.float32)
    z2 = jnp.zeros((NPAD, H), jnp.float32)

    degp, dstp = _sc_prep(s, r, z1)
    dis = _tc_dis(degp)

    xe = _tc_mlp(params["enc"], x_edge, 16)
    p2 = _sc_scatter2(xe, s, r, z2)
    shift = (jnp.asarray(num_nodes, jnp.float32) - float(N)).reshape(1, 1)
    xn, u = _tc_comb0(p2[0], p2[1], shift, dis)

    for i in range(3):
        tx0, u0 = xn, u
        p = _sc_lap(u0, s, dstp, z2)
        tx1, u1 = _tc_hop(p[0], p[1], dis)
        p = _sc_lap(u1, s, dstp, z2)
        tx2, u2 = _tc_hop(p[0], p[1], dis, tx0)
        p = _sc_lap(u2, s, dstp, z2)
        tx3, u3 = _tc_hop(p[0], p[1], dis, tx1)
        p = _sc_lap(u3, s, dstp, z2)
        tx4, _ = _tc_hop(p[0], p[1], dis, tx2)
        xn, u = _tc_cheb_mlp(params["gn"][i], (tx0, tx1, tx2, tx3, tx4), dis)
        gs, gr = _sc_gather2(xn, s, r)
        if i < 2:
            xe = _tc_dec(params["ed"][i], gs, gr, xe)
        else:
            out = _tc_dec(params["ed"][2], gs, gr, xe, final=True)
    return out[:, :3]


# speedup vs baseline: 2.1441x; 1.1479x over previous
import functools

import jax
import jax.numpy as jnp
from jax import lax
from jax.experimental import pallas as pl
from jax.experimental.pallas import tpu as pltpu
from jax.experimental.pallas import tpu_sc as plsc

E = 320000
N = 10000
NPAD = 10240
H = 64
NW = 32
EPW = E // NW
CH = 80
NCHUNK = EPW // CH
RPT = NPAD // 16

@functools.cache
def _mesh():
    return plsc.VectorSubcoreMesh(core_axis_name="c", subcore_axis_name="s")


def _sc_prep_body(s_hbm, r_hbm, z1_hbm, degp_hbm, dstp_hbm,
                  s_v, r_v, w_v, d_v, deg_sh):
    c = lax.axis_index("c")
    t = lax.axis_index("s")
    wid = c * 16 + t
    pltpu.sync_copy(z1_hbm.at[pl.ds(t * RPT, RPT)], deg_sh.at[pl.ds(t * RPT, RPT)])
    plsc.subcore_barrier()

    @pl.loop(0, NCHUNK)
    def _chunk(i):
        base = wid * EPW + i * CH
        pltpu.sync_copy(s_hbm.at[pl.ds(base, CH)], s_v)
        pltpu.sync_copy(r_hbm.at[pl.ds(base, CH)], r_v)

        @pl.loop(0, CH // 16)
        def _sub(j):
            sv = s_v[pl.ds(j * 16, 16)]
            rv = r_v[pl.ds(j * 16, 16)]
            neq = sv != rv
            w_v[pl.ds(j * 16, 16)] = jnp.where(neq, 1.0, 0.0).astype(jnp.float32)
            d_v[pl.ds(j * 16, 16)] = jnp.where(neq, rv, N)

        pltpu.sync_copy(d_v, dstp_hbm.at[pl.ds(base, CH)])
        pltpu.sync_copy(w_v, deg_sh.at[s_v], add=True)

    plsc.subcore_barrier()
    pltpu.sync_copy(deg_sh.at[pl.ds(t * RPT, RPT)], degp_hbm.at[c, pl.ds(t * RPT, RPT)])


def _sc_prep(s, r, z1):
    return pl.kernel(
        _sc_prep_body,
        out_type=(jax.ShapeDtypeStruct((2, NPAD), jnp.float32),
                  jax.ShapeDtypeStruct((E,), jnp.int32)),
        mesh=_mesh(),
        compiler_params=pltpu.CompilerParams(use_tc_tiling_on_sc=False),
        scratch_types=[pltpu.VMEM((CH,), jnp.int32),
                       pltpu.VMEM((CH,), jnp.int32),
                       pltpu.VMEM((CH,), jnp.float32),
                       pltpu.VMEM((CH,), jnp.int32),
                       pltpu.VMEM_SHARED((NPAD,), jnp.float32)],
    )(s, r, z1)


def _sc_scatter2_body(xe_hbm, s_hbm, r_hbm, z2_hbm, out_hbm,
                      s_v, r_v, rows_v, acc_sh):
    c = lax.axis_index("c")
    t = lax.axis_index("s")
    wid = c * 16 + t
    pltpu.sync_copy(z2_hbm.at[pl.ds(t * RPT, RPT)], acc_sh.at[pl.ds(t * RPT, RPT)])
    plsc.subcore_barrier()

    @pl.loop(0, NCHUNK)
    def _chunk(i):
        base = wid * EPW + i * CH
        pltpu.sync_copy(s_hbm.at[pl.ds(base, CH)], s_v)
        pltpu.sync_copy(r_hbm.at[pl.ds(base, CH)], r_v)
        pltpu.sync_copy(xe_hbm.at[pl.ds(base, CH)], rows_v)
        pltpu.sync_copy(rows_v, acc_sh.at[s_v], add=True)
        pltpu.sync_copy(rows_v, acc_sh.at[r_v], add=True)

    plsc.subcore_barrier()
    pltpu.sync_copy(acc_sh.at[pl.ds(t * RPT, RPT)], out_hbm.at[c, pl.ds(t * RPT, RPT)])


def _sc_scatter2(xe, s, r, z2):
    return pl.kernel(
        _sc_scatter2_body,
        out_type=jax.ShapeDtypeStruct((2, NPAD, H), jnp.float32),
        mesh=_mesh(),
        compiler_params=pltpu.CompilerParams(use_tc_tiling_on_sc=False),
        scratch_types=[pltpu.VMEM((CH,), jnp.int32),
                       pltpu.VMEM((CH,), jnp.int32),
                       pltpu.VMEM((CH, H), jnp.float32),
                       pltpu.VMEM_SHARED((NPAD, H), jnp.float32)],
    )(xe, s, r, z2)


def _sc_lap_body(u_hbm, s_hbm, dstp_hbm, z2_hbm, out_hbm,
                 s_a, d_a, rows_a, s_b, d_b, rows_b, acc_sh, sem_a, sem_b):
    c = lax.axis_index("c")
    t = lax.axis_index("s")
    wid = c * 16 + t
    pltpu.sync_copy(z2_hbm.at[pl.ds(t * RPT, RPT)], acc_sh.at[pl.ds(t * RPT, RPT)])
    plsc.subcore_barrier()

    def start(i, s_v, d_v, rows_v, sem):
        base = wid * EPW + i * CH
        pltpu.sync_copy(s_hbm.at[pl.ds(base, CH)], s_v)
        pltpu.sync_copy(dstp_hbm.at[pl.ds(base, CH)], d_v)
        pltpu.async_copy(u_hbm.at[s_v], rows_v, sem)

    def finish(s_v, d_v, rows_v, sem):
        pltpu.make_async_copy(u_hbm.at[s_v], rows_v, sem).wait()
        pltpu.sync_copy(rows_v, acc_sh.at[d_v], add=True)

    start(0, s_a, d_a, rows_a, sem_a)

    @pl.loop(0, (NCHUNK - 1) // 2)
    def _chunk(j):
        i = 1 + 2 * j
        start(i, s_b, d_b, rows_b, sem_b)
        finish(s_a, d_a, rows_a, sem_a)
        start(i + 1, s_a, d_a, rows_a, sem_a)
        finish(s_b, d_b, rows_b, sem_b)

    finish(s_a, d_a, rows_a, sem_a)
    plsc.subcore_barrier()
    pltpu.sync_copy(acc_sh.at[pl.ds(t * RPT, RPT)], out_hbm.at[c, pl.ds(t * RPT, RPT)])


def _sc_lap(u, s, dstp, z2):
    return pl.kernel(
        _sc_lap_body,
        out_type=jax.ShapeDtypeStruct((2, NPAD, H), jnp.float32),
        mesh=_mesh(),
        compiler_params=pltpu.CompilerParams(use_tc_tiling_on_sc=False),
        scratch_types=[pltpu.VMEM((CH,), jnp.int32),
                       pltpu.VMEM((CH,), jnp.int32),
                       pltpu.VMEM((CH, H), jnp.float32),
                       pltpu.VMEM((CH,), jnp.int32),
                       pltpu.VMEM((CH,), jnp.int32),
                       pltpu.VMEM((CH, H), jnp.float32),
                       pltpu.VMEM_SHARED((NPAD, H), jnp.float32),
                       pltpu.SemaphoreType.DMA,
                       pltpu.SemaphoreType.DMA],
    )(u, s, dstp, z2)


def _sc_gather2_body(xn_hbm, s_hbm, r_hbm, gs_hbm, gr_hbm,
                     s_v, r_v, rows_v, sem):
    c = lax.axis_index("c")
    t = lax.axis_index("s")
    wid = c * 16 + t

    @pl.loop(0, NCHUNK)
    def _chunk(i):
        base = wid * EPW + i * CH
        pltpu.sync_copy(s_hbm.at[pl.ds(base, CH)], s_v)
        pltpu.sync_copy(r_hbm.at[pl.ds(base, CH)], r_v)
        pltpu.async_copy(xn_hbm.at[s_v], rows_v, sem).wait()
        pltpu.sync_copy(rows_v, gs_hbm.at[pl.ds(base, CH)])
        pltpu.async_copy(xn_hbm.at[r_v], rows_v, sem).wait()
        pltpu.sync_copy(rows_v, gr_hbm.at[pl.ds(base, CH)])


def _sc_gather2(xn, s, r):
    return pl.kernel(
        _sc_gather2_body,
        out_type=(jax.ShapeDtypeStruct((E, H), jnp.float32),
                  jax.ShapeDtypeStruct((E, H), jnp.float32)),
        mesh=_mesh(),
        compiler_params=pltpu.CompilerParams(use_tc_tiling_on_sc=False),
        scratch_types=[pltpu.VMEM((CH,), jnp.int32),
                       pltpu.VMEM((CH,), jnp.int32),
                       pltpu.VMEM((CH, H), jnp.float32),
                       pltpu.SemaphoreType.DMA],
    )(xn, s, r)


_F32 = functools.partial(jnp.dot, preferred_element_type=jnp.float32)
EB = 2000
NB = 2048


def _xsum64(h):
    p = h[:, 0:8] + h[:, 8:16]
    for k in range(2, 8):
        p = p + h[:, 8 * k:8 * k + 8]
    q = p[:, 0:4] + p[:, 4:8]
    r = q[:, 0:2] + q[:, 2:4]
    return r[:, 0:1] + r[:, 1:2]


def _layer_norm(h, g, b, width=64.0):
    if width == 3.0:
        mu = ((h[:, 0:1] + h[:, 1:2]) + h[:, 2:3]) / 3.0
        d = h - mu
        var = ((d[:, 0:1] * d[:, 0:1] + d[:, 1:2] * d[:, 1:2])
               + d[:, 2:3] * d[:, 2:3]) / 3.0
    else:
        mu = _xsum64(h) / 64.0
        d = h - mu
        var = _xsum64(d * d) / 64.0
    return d * lax.rsqrt(var + 1e-5) * g + b


def _mlp_tail(h0, w1, b1, w2, b2, w3, b3, g, b, width=float(H)):
    h = jnp.maximum(h0, 0.0)
    h = jnp.maximum(_F32(h, w1[...]) + b1[...], 0.0)
    h = jnp.maximum(_F32(h, w2[...]) + b2[...], 0.0)
    h = _F32(h, w3[...]) + b3[...]
    return _layer_norm(h, g[...], b[...], width)


def _tc_mlp_body(x, w0, b0, w1, b1, w2, b2, w3, b3, g, b, o):
    h0 = _F32(x[...], w0[...]) + b0[...]
    o[...] = _mlp_tail(h0, w1, b1, w2, b2, w3, b3, g, b)


def _tc_mlp(p, x, win):
    nblk = x.shape[0] // EB
    args = (x, p["l0"]["w"], p["l0"]["b"].reshape(1, H),
            p["l1"]["w"], p["l1"]["b"].reshape(1, H),
            p["l2"]["w"], p["l2"]["b"].reshape(1, H),
            p["l3"]["w"], p["l3"]["b"].reshape(1, H),
            p["ln_g"].reshape(1, H), p["ln_b"].reshape(1, H))
    wspec = [pl.BlockSpec(a.shape, lambda i: (0, 0)) for a in args[1:]]
    return pl.pallas_call(
        _tc_mlp_body,
        grid=(nblk,),
        in_specs=[pl.BlockSpec((EB, win), lambda i: (i, 0))] + wspec,
        out_specs=pl.BlockSpec((EB, H), lambda i: (i, 0)),
        out_shape=jax.ShapeDtypeStruct((x.shape[0], H), jnp.float32),
    )(*args)


def _tc_dec_body(gs, gr, xe, w0, b0, w1, b1, w2, b2, w3, b3, g, b, o):
    cat = jnp.concatenate([gs[...], gr[...], xe[...]], axis=1)
    h0 = _F32(cat, w0[...]) + b0[...]
    o[...] = _mlp_tail(h0, w1, b1, w2, b2, w3, b3, g, b)


def _tc_dec_final_body(gs, gr, xe, w0, b0, w1, b1, w2, b2, w3, b3, g, b, o):
    cat = jnp.concatenate([gs[...], gr[...], xe[...]], axis=1)
    h0 = _F32(cat, w0[...]) + b0[...]
    o[...] = _mlp_tail(h0, w1, b1, w2, b2, w3, b3, g, b, width=3.0)


def _pad8(a):
    out = jnp.zeros(a.shape[:-1] + (8,), a.dtype)
    return out.at[..., :a.shape[-1]].set(a)


def _tc_dec(p, gs, gr, xe, final=False):
    w0 = p["l0"]["w"]
    if final:
        rest = (_pad8(w0), _pad8(p["l0"]["b"]).reshape(1, 8),
                _pad8(_pad8(p["l1"]["w"]).T).T, _pad8(p["l1"]["b"]).reshape(1, 8),
                _pad8(_pad8(p["l2"]["w"]).T).T, _pad8(p["l2"]["b"]).reshape(1, 8),
                _pad8(_pad8(p["l3"]["w"]).T).T, _pad8(p["l3"]["b"]).reshape(1, 8),
                _pad8(p["ln_g"]).reshape(1, 8), _pad8(p["ln_b"]).reshape(1, 8))
        width, body = 8, _tc_dec_final_body
    else:
        rest = (w0, p["l0"]["b"].reshape(1, H),
                p["l1"]["w"], p["l1"]["b"].reshape(1, H),
                p["l2"]["w"], p["l2"]["b"].reshape(1, H),
                p["l3"]["w"], p["l3"]["b"].reshape(1, H),
                p["ln_g"].reshape(1, H), p["ln_b"].reshape(1, H))
        width, body = H, _tc_dec_body
    args = (gs, gr, xe) + rest
    wspec = [pl.BlockSpec(a.shape, lambda i: (0, 0)) for a in args[3:]]
    return pl.pallas_call(
        body,
        grid=(E // EB,),
        in_specs=[pl.BlockSpec((EB, H), lambda i: (i, 0))] * 3 + wspec,
        out_specs=pl.BlockSpec((EB, width), lambda i: (i, 0)),
        out_shape=jax.ShapeDtypeStruct((E, width), jnp.float32),
    )(*args)


def _tc_dis_body(d0, d1, o):
    deg = d0[...] + d1[...]
    o[...] = jnp.where(deg > 0, 1.0 / jnp.sqrt(jnp.maximum(deg, 1e-12)), 0.0)


def _tc_dis(degp):
    flat = degp.reshape(2, 80, 128)
    out = pl.pallas_call(
        _tc_dis_body,
        in_specs=[pl.BlockSpec((80, 128), lambda: (0, 0))] * 2,
        out_specs=pl.BlockSpec((80, 128), lambda: (0, 0)),
        out_shape=jax.ShapeDtypeStruct((80, 128), jnp.float32),
    )(flat[0], flat[1])
    return out.reshape(NPAD, 1)


def _tc_comb0_body(p0, p1, shift, dis, xn_o, u_o):
    xn = p0[...] + p1[...] + shift[...]
    xn_o[...] = xn
    u_o[...] = xn * dis[...]


def _tc_comb0(p0, p1, shift, dis):
    return pl.pallas_call(
        _tc_comb0_body,
        grid=(NPAD // NB,),
        in_specs=[pl.BlockSpec((NB, H), lambda i: (i, 0)),
                  pl.BlockSpec((NB, H), lambda i: (i, 0)),
                  pl.BlockSpec((1, 1), lambda i: (0, 0)),
                  pl.BlockSpec((NB, 1), lambda i: (i, 0))],
        out_specs=[pl.BlockSpec((NB, H), lambda i: (i, 0))] * 2,
        out_shape=(jax.ShapeDtypeStruct((NPAD, H), jnp.float32),
                   jax.ShapeDtypeStruct((NPAD, H), jnp.float32)),
    )(p0, p1, shift, dis)


def _tc_hop1_body(p0, p1, dis, tx_o, u_o):
    t = -(p0[...] + p1[...]) * dis[...]
    tx_o[...] = t
    u_o[...] = t * dis[...]


def _tc_hopk_body(p0, p1, txpp, dis, tx_o, u_o):
    t = -2.0 * (p0[...] + p1[...]) * dis[...] - txpp[...]
    tx_o[...] = t
    u_o[...] = t * dis[...]


def _tc_hop(p0, p1, dis, txpp=None):
    body = _tc_hop1_body if txpp is None else _tc_hopk_body
    extra = [] if txpp is None else [txpp]
    nspec = pl.BlockSpec((NB, H), lambda i: (i, 0))
    return pl.pallas_call(
        body,
        grid=(NPAD // NB,),
        in_specs=[nspec, nspec] + [nspec] * len(extra)
        + [pl.BlockSpec((NB, 1), lambda i: (i, 0))],
        out_specs=[nspec] * 2,
        out_shape=(jax.ShapeDtypeStruct((NPAD, H), jnp.float32),
                   jax.ShapeDtypeStruct((NPAD, H), jnp.float32)),
    )(p0, p1, *extra, dis)


def _tc_cheb_mlp_body(t0, t1, t2, t3, t4, c0, c1, c2, c3, c4, cb,
                      w0, b0, w1, b1, w2, b2, w3, b3, g, b, dis, xn_o, u_o):
    acc = (_F32(t0[...], c0[...]) + _F32(t1[...], c1[...])
           + _F32(t2[...], c2[...]) + _F32(t3[...], c3[...])
           + _F32(t4[...], c4[...]) + cb[...])
    h0 = _F32(acc, w0[...]) + b0[...]
    xn = _mlp_tail(h0, w1, b1, w2, b2, w3, b3, g, b)
    xn_o[...] = xn
    u_o[...] = xn * dis[...]


def _tc_cheb_mlp(gn, txs, dis):
    cw, m = gn["cheb"], gn["mlp"]
    args = tuple(txs) + tuple(cw["w"][k] for k in range(5)) + (
        cw["b"].reshape(1, H),
        m["l0"]["w"], m["l0"]["b"].reshape(1, H),
        m["l1"]["w"], m["l1"]["b"].reshape(1, H),
        m["l2"]["w"], m["l2"]["b"].reshape(1, H),
        m["l3"]["w"], m["l3"]["b"].reshape(1, H),
        m["ln_g"].reshape(1, H), m["ln_b"].reshape(1, H), dis)
    nspec = pl.BlockSpec((NB, H), lambda i: (i, 0))
    wspec = [pl.BlockSpec(a.shape, lambda i: (0, 0)) for a in args[5:-1]]
    return pl.pallas_call(
        _tc_cheb_mlp_body,
        grid=(NPAD // NB,),
        in_specs=[nspec] * 5 + wspec + [pl.BlockSpec((NB, 1), lambda i: (i, 0))],
        out_specs=[nspec] * 2,
        out_shape=(jax.ShapeDtypeStruct((NPAD, H), jnp.float32),
                   jax.ShapeDtypeStruct((NPAD, H), jnp.float32)),
    )(*args)


def kernel(x_edge, edge_index, num_nodes, params):
    s = edge_index[:, 0]
    r = edge_index[:, 1]
    z1 = jnp.zeros((NPAD,), jnp.float32)
    z2 = jnp.zeros((NPAD, H), jnp.float32)

    degp, dstp = _sc_prep(s, r, z1)
    dis = _tc_dis(degp)

    xe = _tc_mlp(params["enc"], x_edge, 16)
    p2 = _sc_scatter2(xe, s, r, z2)
    shift = (jnp.asarray(num_nodes, jnp.float32) - float(N)).reshape(1, 1)
    xn, u = _tc_comb0(p2[0], p2[1], shift, dis)

    for i in range(3):
        tx0, u0 = xn, u
        p = _sc_lap(u0, s, dstp, z2)
        tx1, u1 = _tc_hop(p[0], p[1], dis)
        p = _sc_lap(u1, s, dstp, z2)
        tx2, u2 = _tc_hop(p[0], p[1], dis, tx0)
        p = _sc_lap(u2, s, dstp, z2)
        tx3, u3 = _tc_hop(p[0], p[1], dis, tx1)
        p = _sc_lap(u3, s, dstp, z2)
        tx4, _ = _tc_hop(p[0], p[1], dis, tx2)
        xn, u = _tc_cheb_mlp(params["gn"][i], (tx0, tx1, tx2, tx3, tx4), dis)
        gs, gr = _sc_gather2(xn, s, r)
        if i < 2:
            xe = _tc_dec(params["ed"][i], gs, gr, xe)
        else:
            out = _tc_dec(params["ed"][2], gs, gr, xe, final=True)
    return out[:, :3]


# speedup vs baseline: 2.1826x; 1.0180x over previous
import functools

import jax
import jax.numpy as jnp
from jax import lax
from jax.experimental import pallas as pl
from jax.experimental.pallas import tpu as pltpu
from jax.experimental.pallas import tpu_sc as plsc

E = 320000
N = 10000
NPAD = 10240
H = 64
NW = 32
EPW = E // NW
CH = 80
NCHUNK = EPW // CH
RPT = NPAD // 16

@functools.cache
def _mesh():
    return plsc.VectorSubcoreMesh(core_axis_name="c", subcore_axis_name="s")


def _sc_prep_body(s_hbm, r_hbm, z1_hbm, degp_hbm, dstp_hbm,
                  s_v, r_v, w_v, d_v, deg_sh):
    c = lax.axis_index("c")
    t = lax.axis_index("s")
    wid = c * 16 + t
    pltpu.sync_copy(z1_hbm.at[pl.ds(t * RPT, RPT)], deg_sh.at[pl.ds(t * RPT, RPT)])
    plsc.subcore_barrier()

    @pl.loop(0, NCHUNK)
    def _chunk(i):
        base = wid * EPW + i * CH
        pltpu.sync_copy(s_hbm.at[pl.ds(base, CH)], s_v)
        pltpu.sync_copy(r_hbm.at[pl.ds(base, CH)], r_v)

        @pl.loop(0, CH // 16)
        def _sub(j):
            sv = s_v[pl.ds(j * 16, 16)]
            rv = r_v[pl.ds(j * 16, 16)]
            neq = sv != rv
            w_v[pl.ds(j * 16, 16)] = jnp.where(neq, 1.0, 0.0).astype(jnp.float32)
            d_v[pl.ds(j * 16, 16)] = jnp.where(neq, rv, N)

        pltpu.sync_copy(d_v, dstp_hbm.at[pl.ds(base, CH)])
        pltpu.sync_copy(w_v, deg_sh.at[s_v], add=True)

    plsc.subcore_barrier()
    pltpu.sync_copy(deg_sh.at[pl.ds(t * RPT, RPT)], degp_hbm.at[c, pl.ds(t * RPT, RPT)])


def _sc_prep(s, r, z1):
    return pl.kernel(
        _sc_prep_body,
        out_type=(jax.ShapeDtypeStruct((2, NPAD), jnp.float32),
                  jax.ShapeDtypeStruct((E,), jnp.int32)),
        mesh=_mesh(),
        compiler_params=pltpu.CompilerParams(use_tc_tiling_on_sc=False),
        scratch_types=[pltpu.VMEM((CH,), jnp.int32),
                       pltpu.VMEM((CH,), jnp.int32),
                       pltpu.VMEM((CH,), jnp.float32),
                       pltpu.VMEM((CH,), jnp.int32),
                       pltpu.VMEM_SHARED((NPAD,), jnp.float32)],
    )(s, r, z1)


def _sc_scatter2_body(xe_hbm, s_hbm, r_hbm, z2_hbm, out_hbm,
                      s_v, r_v, rows_v, acc_sh):
    c = lax.axis_index("c")
    t = lax.axis_index("s")
    wid = c * 16 + t
    pltpu.sync_copy(z2_hbm.at[pl.ds(t * RPT, RPT)], acc_sh.at[pl.ds(t * RPT, RPT)])
    plsc.subcore_barrier()

    @pl.loop(0, NCHUNK)
    def _chunk(i):
        base = wid * EPW + i * CH
        pltpu.sync_copy(s_hbm.at[pl.ds(base, CH)], s_v)
        pltpu.sync_copy(r_hbm.at[pl.ds(base, CH)], r_v)
        pltpu.sync_copy(xe_hbm.at[pl.ds(base, CH)], rows_v)
        pltpu.sync_copy(rows_v, acc_sh.at[s_v], add=True)
        pltpu.sync_copy(rows_v, acc_sh.at[r_v], add=True)

    plsc.subcore_barrier()
    pltpu.sync_copy(acc_sh.at[pl.ds(t * RPT, RPT)], out_hbm.at[c, pl.ds(t * RPT, RPT)])


def _sc_scatter2(xe, s, r, z2):
    return pl.kernel(
        _sc_scatter2_body,
        out_type=jax.ShapeDtypeStruct((2, NPAD, H), jnp.float32),
        mesh=_mesh(),
        compiler_params=pltpu.CompilerParams(use_tc_tiling_on_sc=False),
        scratch_types=[pltpu.VMEM((CH,), jnp.int32),
                       pltpu.VMEM((CH,), jnp.int32),
                       pltpu.VMEM((CH, H), jnp.float32),
                       pltpu.VMEM_SHARED((NPAD, H), jnp.float32)],
    )(xe, s, r, z2)


def _sc_lap_body(u_hbm, s_hbm, dstp_hbm, z2_hbm, out_hbm,
                 s_a, d_a, rows_a, s_b, d_b, rows_b, acc_sh, sem_a, sem_b):
    c = lax.axis_index("c")
    t = lax.axis_index("s")
    wid = c * 16 + t
    pltpu.sync_copy(z2_hbm.at[pl.ds(t * RPT, RPT)], acc_sh.at[pl.ds(t * RPT, RPT)])
    plsc.subcore_barrier()

    def start(i, s_v, d_v, rows_v, sem):
        base = wid * EPW + i * CH
        pltpu.sync_copy(s_hbm.at[pl.ds(base, CH)], s_v)
        pltpu.sync_copy(dstp_hbm.at[pl.ds(base, CH)], d_v)
        pltpu.async_copy(u_hbm.at[s_v], rows_v, sem)

    def finish(s_v, d_v, rows_v, sem):
        pltpu.make_async_copy(u_hbm.at[s_v], rows_v, sem).wait()
        pltpu.sync_copy(rows_v, acc_sh.at[d_v], add=True)

    start(0, s_a, d_a, rows_a, sem_a)

    @pl.loop(0, (NCHUNK - 1) // 2)
    def _chunk(j):
        i = 1 + 2 * j
        start(i, s_b, d_b, rows_b, sem_b)
        finish(s_a, d_a, rows_a, sem_a)
        start(i + 1, s_a, d_a, rows_a, sem_a)
        finish(s_b, d_b, rows_b, sem_b)

    finish(s_a, d_a, rows_a, sem_a)
    plsc.subcore_barrier()
    pltpu.sync_copy(acc_sh.at[pl.ds(t * RPT, RPT)], out_hbm.at[c, pl.ds(t * RPT, RPT)])


def _sc_lap(u, s, dstp, z2):
    return pl.kernel(
        _sc_lap_body,
        out_type=jax.ShapeDtypeStruct((2, NPAD, H), jnp.float32),
        mesh=_mesh(),
        compiler_params=pltpu.CompilerParams(use_tc_tiling_on_sc=False),
        scratch_types=[pltpu.VMEM((CH,), jnp.int32),
                       pltpu.VMEM((CH,), jnp.int32),
                       pltpu.VMEM((CH, H), jnp.float32),
                       pltpu.VMEM((CH,), jnp.int32),
                       pltpu.VMEM((CH,), jnp.int32),
                       pltpu.VMEM((CH, H), jnp.float32),
                       pltpu.VMEM_SHARED((NPAD, H), jnp.float32),
                       pltpu.SemaphoreType.DMA,
                       pltpu.SemaphoreType.DMA],
    )(u, s, dstp, z2)


def _sc_gather2_body(xn_hbm, s_hbm, r_hbm, gs_hbm, gr_hbm,
                     s_a, r_a, rs_a, rr_a, s_b, r_b, rs_b, rr_b,
                     sem_a, sem_b):
    c = lax.axis_index("c")
    t = lax.axis_index("s")
    wid = c * 16 + t

    def start(i, s_v, r_v, rs_v, rr_v, sem):
        base = wid * EPW + i * CH
        pltpu.sync_copy(s_hbm.at[pl.ds(base, CH)], s_v)
        pltpu.sync_copy(r_hbm.at[pl.ds(base, CH)], r_v)
        pltpu.async_copy(xn_hbm.at[s_v], rs_v, sem)
        pltpu.async_copy(xn_hbm.at[r_v], rr_v, sem)

    def finish(i, s_v, r_v, rs_v, rr_v, sem):
        base = wid * EPW + i * CH
        pltpu.make_async_copy(xn_hbm.at[s_v], rs_v, sem).wait()
        pltpu.make_async_copy(xn_hbm.at[r_v], rr_v, sem).wait()
        pltpu.sync_copy(rs_v, gs_hbm.at[pl.ds(base, CH)])
        pltpu.sync_copy(rr_v, gr_hbm.at[pl.ds(base, CH)])

    start(0, s_a, r_a, rs_a, rr_a, sem_a)

    @pl.loop(0, (NCHUNK - 1) // 2)
    def _chunk(j):
        i = 1 + 2 * j
        start(i, s_b, r_b, rs_b, rr_b, sem_b)
        finish(i - 1, s_a, r_a, rs_a, rr_a, sem_a)
        start(i + 1, s_a, r_a, rs_a, rr_a, sem_a)
        finish(i, s_b, r_b, rs_b, rr_b, sem_b)

    finish(NCHUNK - 1, s_a, r_a, rs_a, rr_a, sem_a)


def _sc_gather2(xn, s, r):
    return pl.kernel(
        _sc_gather2_body,
        out_type=(jax.ShapeDtypeStruct((E, H), jnp.float32),
                  jax.ShapeDtypeStruct((E, H), jnp.float32)),
        mesh=_mesh(),
        compiler_params=pltpu.CompilerParams(use_tc_tiling_on_sc=False),
        scratch_types=[pltpu.VMEM((CH,), jnp.int32),
                       pltpu.VMEM((CH,), jnp.int32),
                       pltpu.VMEM((CH, H), jnp.float32),
                       pltpu.VMEM((CH, H), jnp.float32),
                       pltpu.VMEM((CH,), jnp.int32),
                       pltpu.VMEM((CH,), jnp.int32),
                       pltpu.VMEM((CH, H), jnp.float32),
                       pltpu.VMEM((CH, H), jnp.float32),
                       pltpu.SemaphoreType.DMA,
                       pltpu.SemaphoreType.DMA],
    )(xn, s, r)


_F32 = functools.partial(jnp.dot, preferred_element_type=jnp.float32)
EB = 2000
NB = 2048


def _xsum64(h):
    p = h[:, 0:8] + h[:, 8:16]
    for k in range(2, 8):
        p = p + h[:, 8 * k:8 * k + 8]
    q = p[:, 0:4] + p[:, 4:8]
    r = q[:, 0:2] + q[:, 2:4]
    return r[:, 0:1] + r[:, 1:2]


def _layer_norm(h, g, b, width=64.0):
    if width == 3.0:
        mu = ((h[:, 0:1] + h[:, 1:2]) + h[:, 2:3]) / 3.0
        d = h - mu
        var = ((d[:, 0:1] * d[:, 0:1] + d[:, 1:2] * d[:, 1:2])
               + d[:, 2:3] * d[:, 2:3]) / 3.0
    else:
        mu = _xsum64(h) / 64.0
        d = h - mu
        var = _xsum64(d * d) / 64.0
    return d * lax.rsqrt(var + 1e-5) * g + b


def _mlp_tail(h0, w1, b1, w2, b2, w3, b3, g, b, width=float(H)):
    h = jnp.maximum(h0, 0.0)
    h = jnp.maximum(_F32(h, w1[...]) + b1[...], 0.0)
    h = jnp.maximum(_F32(h, w2[...]) + b2[...], 0.0)
    h = _F32(h, w3[...]) + b3[...]
    return _layer_norm(h, g[...], b[...], width)


def _tc_mlp_body(x, w0, b0, w1, b1, w2, b2, w3, b3, g, b, o):
    h0 = _F32(x[...], w0[...]) + b0[...]
    o[...] = _mlp_tail(h0, w1, b1, w2, b2, w3, b3, g, b)


def _tc_mlp(p, x, win):
    nblk = x.shape[0] // EB
    args = (x, p["l0"]["w"], p["l0"]["b"].reshape(1, H),
            p["l1"]["w"], p["l1"]["b"].reshape(1, H),
            p["l2"]["w"], p["l2"]["b"].reshape(1, H),
            p["l3"]["w"], p["l3"]["b"].reshape(1, H),
            p["ln_g"].reshape(1, H), p["ln_b"].reshape(1, H))
    wspec = [pl.BlockSpec(a.shape, lambda i: (0, 0)) for a in args[1:]]
    return pl.pallas_call(
        _tc_mlp_body,
        grid=(nblk,),
        in_specs=[pl.BlockSpec((EB, win), lambda i: (i, 0))] + wspec,
        out_specs=pl.BlockSpec((EB, H), lambda i: (i, 0)),
        out_shape=jax.ShapeDtypeStruct((x.shape[0], H), jnp.float32),
    )(*args)


def _tc_dec_body(gs, gr, xe, w0, b0, w1, b1, w2, b2, w3, b3, g, b, o):
    cat = jnp.concatenate([gs[...], gr[...], xe[...]], axis=1)
    h0 = _F32(cat, w0[...]) + b0[...]
    o[...] = _mlp_tail(h0, w1, b1, w2, b2, w3, b3, g, b)


def _tc_dec_final_body(gs, gr, xe, w0, b0, w1, b1, w2, b2, w3, b3, g, b, o):
    cat = jnp.concatenate([gs[...], gr[...], xe[...]], axis=1)
    h0 = _F32(cat, w0[...]) + b0[...]
    o[...] = _mlp_tail(h0, w1, b1, w2, b2, w3, b3, g, b, width=3.0)


def _pad8(a):
    out = jnp.zeros(a.shape[:-1] + (8,), a.dtype)
    return out.at[..., :a.shape[-1]].set(a)


def _tc_dec(p, gs, gr, xe, final=False):
    w0 = p["l0"]["w"]
    if final:
        rest = (_pad8(w0), _pad8(p["l0"]["b"]).reshape(1, 8),
                _pad8(_pad8(p["l1"]["w"]).T).T, _pad8(p["l1"]["b"]).reshape(1, 8),
                _pad8(_pad8(p["l2"]["w"]).T).T, _pad8(p["l2"]["b"]).reshape(1, 8),
                _pad8(_pad8(p["l3"]["w"]).T).T, _pad8(p["l3"]["b"]).reshape(1, 8),
                _pad8(p["ln_g"]).reshape(1, 8), _pad8(p["ln_b"]).reshape(1, 8))
        width, body = 8, _tc_dec_final_body
    else:
        rest = (w0, p["l0"]["b"].reshape(1, H),
                p["l1"]["w"], p["l1"]["b"].reshape(1, H),
                p["l2"]["w"], p["l2"]["b"].reshape(1, H),
                p["l3"]["w"], p["l3"]["b"].reshape(1, H),
                p["ln_g"].reshape(1, H), p["ln_b"].reshape(1, H))
        width, body = H, _tc_dec_body
    args = (gs, gr, xe) + rest
    wspec = [pl.BlockSpec(a.shape, lambda i: (0, 0)) for a in args[3:]]
    return pl.pallas_call(
        body,
        grid=(E // EB,),
        in_specs=[pl.BlockSpec((EB, H), lambda i: (i, 0))] * 3 + wspec,
        out_specs=pl.BlockSpec((EB, width), lambda i: (i, 0)),
        out_shape=jax.ShapeDtypeStruct((E, width), jnp.float32),
    )(*args)


def _tc_dis_body(d0, d1, o):
    deg = d0[...] + d1[...]
    o[...] = jnp.where(deg > 0, 1.0 / jnp.sqrt(jnp.maximum(deg, 1e-12)), 0.0)


def _tc_dis(degp):
    flat = degp.reshape(2, 80, 128)
    out = pl.pallas_call(
        _tc_dis_body,
        in_specs=[pl.BlockSpec((80, 128), lambda: (0, 0))] * 2,
        out_specs=pl.BlockSpec((80, 128), lambda: (0, 0)),
        out_shape=jax.ShapeDtypeStruct((80, 128), jnp.float32),
    )(flat[0], flat[1])
    return out.reshape(NPAD, 1)


def _tc_comb0_body(p0, p1, shift, dis, xn_o, u_o):
    xn = p0[...] + p1[...] + shift[...]
    xn_o[...] = xn
    u_o[...] = xn * dis[...]


def _tc_comb0(p0, p1, shift, dis):
    return pl.pallas_call(
        _tc_comb0_body,
        grid=(NPAD // NB,),
        in_specs=[pl.BlockSpec((NB, H), lambda i: (i, 0)),
                  pl.BlockSpec((NB, H), lambda i: (i, 0)),
                  pl.BlockSpec((1, 1), lambda i: (0, 0)),
                  pl.BlockSpec((NB, 1), lambda i: (i, 0))],
        out_specs=[pl.BlockSpec((NB, H), lambda i: (i, 0))] * 2,
        out_shape=(jax.ShapeDtypeStruct((NPAD, H), jnp.float32),
                   jax.ShapeDtypeStruct((NPAD, H), jnp.float32)),
    )(p0, p1, shift, dis)


def _tc_hop1_body(p0, p1, dis, tx_o, u_o):
    t = -(p0[...] + p1[...]) * dis[...]
    tx_o[...] = t
    u_o[...] = t * dis[...]


def _tc_hopk_body(p0, p1, txpp, dis, tx_o, u_o):
    t = -2.0 * (p0[...] + p1[...]) * dis[...] - txpp[...]
    tx_o[...] = t
    u_o[...] = t * dis[...]


def _tc_hop(p0, p1, dis, txpp=None):
    body = _tc_hop1_body if txpp is None else _tc_hopk_body
    extra = [] if txpp is None else [txpp]
    nspec = pl.BlockSpec((NB, H), lambda i: (i, 0))
    return pl.pallas_call(
        body,
        grid=(NPAD // NB,),
        in_specs=[nspec, nspec] + [nspec] * len(extra)
        + [pl.BlockSpec((NB, 1), lambda i: (i, 0))],
        out_specs=[nspec] * 2,
        out_shape=(jax.ShapeDtypeStruct((NPAD, H), jnp.float32),
                   jax.ShapeDtypeStruct((NPAD, H), jnp.float32)),
    )(p0, p1, *extra, dis)


def _tc_cheb_mlp_body(t0, t1, t2, t3, t4, c0, c1, c2, c3, c4, cb,
                      w0, b0, w1, b1, w2, b2, w3, b3, g, b, dis, xn_o, u_o):
    acc = (_F32(t0[...], c0[...]) + _F32(t1[...], c1[...])
           + _F32(t2[...], c2[...]) + _F32(t3[...], c3[...])
           + _F32(t4[...], c4[...]) + cb[...])
    h0 = _F32(acc, w0[...]) + b0[...]
    xn = _mlp_tail(h0, w1, b1, w2, b2, w3, b3, g, b)
    xn_o[...] = xn
    u_o[...] = xn * dis[...]


def _tc_cheb_mlp(gn, txs, dis):
    cw, m = gn["cheb"], gn["mlp"]
    args = tuple(txs) + tuple(cw["w"][k] for k in range(5)) + (
        cw["b"].reshape(1, H),
        m["l0"]["w"], m["l0"]["b"].reshape(1, H),
        m["l1"]["w"], m["l1"]["b"].reshape(1, H),
        m["l2"]["w"], m["l2"]["b"].reshape(1, H),
        m["l3"]["w"], m["l3"]["b"].reshape(1, H),
        m["ln_g"].reshape(1, H), m["ln_b"].reshape(1, H), dis)
    nspec = pl.BlockSpec((NB, H), lambda i: (i, 0))
    wspec = [pl.BlockSpec(a.shape, lambda i: (0, 0)) for a in args[5:-1]]
    return pl.pallas_call(
        _tc_cheb_mlp_body,
        grid=(NPAD // NB,),
        in_specs=[nspec] * 5 + wspec + [pl.BlockSpec((NB, 1), lambda i: (i, 0))],
        out_specs=[nspec] * 2,
        out_shape=(jax.ShapeDtypeStruct((NPAD, H), jnp.float32),
                   jax.ShapeDtypeStruct((NPAD, H), jnp.float32)),
    )(*args)


def kernel(x_edge, edge_index, num_nodes, params):
    s = edge_index[:, 0]
    r = edge_index[:, 1]
    z1 = jnp.zeros((NPAD,), jnp.float32)
    z2 = jnp.zeros((NPAD, H), jnp.float32)

    degp, dstp = _sc_prep(s, r, z1)
    dis = _tc_dis(degp)

    xe = _tc_mlp(params["enc"], x_edge, 16)
    p2 = _sc_scatter2(xe, s, r, z2)
    shift = (jnp.asarray(num_nodes, jnp.float32) - float(N)).reshape(1, 1)
    xn, u = _tc_comb0(p2[0], p2[1], shift, dis)

    for i in range(3):
        tx0, u0 = xn, u
        p = _sc_lap(u0, s, dstp, z2)
        tx1, u1 = _tc_hop(p[0], p[1], dis)
        p = _sc_lap(u1, s, dstp, z2)
        tx2, u2 = _tc_hop(p[0], p[1], dis, tx0)
        p = _sc_lap(u2, s, dstp, z2)
        tx3, u3 = _tc_hop(p[0], p[1], dis, tx1)
        p = _sc_lap(u3, s, dstp, z2)
        tx4, _ = _tc_hop(p[0], p[1], dis, tx2)
        xn, u = _tc_cheb_mlp(params["gn"][i], (tx0, tx1, tx2, tx3, tx4), dis)
        gs, gr = _sc_gather2(xn, s, r)
        if i < 2:
            xe = _tc_dec(params["ed"][i], gs, gr, xe)
        else:
            out = _tc_dec(params["ed"][2], gs, gr, xe, final=True)
    return out[:, :3]
